# baseline (speedup 1.0000x reference)
import functools

import jax
import jax.numpy as jnp
from jax import lax
from jax.experimental import pallas as pl
from jax.experimental.pallas import tpu as pltpu
from jax.experimental.pallas import tpu_sc as plsc

B, S = 4, 2048
V, D = 1024, 768
EPS = 1e-12

NC, NS, L = 2, 16, 16
NW = NC * NS
N = B * S
NTOK = N // NW
T = 8
NCHUNK = NTOK // T
RPC = 6 * T
NJ = D // L


def _sc_body(table_hbm, bboxT_hbm, w_hbm, b_hbm, out_hbm,
             bbox_v, idx_v, rows0, rows1, out0, out1, w_v, b_v,
             gsem0, gsem1, osem0, osem1):
    wid = lax.axis_index("s") * NC + lax.axis_index("c")
    base = wid * NTOK

    for g in range(4):
        pltpu.sync_copy(bboxT_hbm.at[g, pl.ds(base, NTOK)], bbox_v.at[g])
    pltpu.sync_copy(w_hbm, w_v)
    pltpu.sync_copy(b_hbm, b_v)

    for i in range(NTOK // L):
        t0 = i * L
        b0 = bbox_v[0, pl.ds(t0, L)]
        b1 = bbox_v[1, pl.ds(t0, L)]
        b2 = bbox_v[2, pl.ds(t0, L)]
        b3 = bbox_v[3, pl.ds(t0, L)]
        vals = (b0, b1 + V, b2, b3 + V, (b3 - b1) + 2 * V, (b2 - b0) + 3 * V)
        for g in range(6):
            idx_v[g, pl.ds(t0, L)] = vals[g]

    def gather_start(c, rows, sem):
        for g in range(6):
            pltpu.make_async_copy(
                table_hbm.at[idx_v.at[g, pl.ds(c * T, T)]],
                rows.at[pl.ds(g * T, T)], sem).start()

    def gather_wait(c, rows, sem):
        for g in range(6):
            pltpu.make_async_copy(
                table_hbm.at[idx_v.at[g, pl.ds(c * T, T)]],
                rows.at[pl.ds(g * T, T)], sem).wait()

    def out_start(c, outb, sem):
        pltpu.make_async_copy(
            outb, out_hbm.at[pl.ds(base + c * T, T)], sem).start()

    def out_wait(c, outb, sem):
        pltpu.make_async_copy(
            outb, out_hbm.at[pl.ds(base + c * T, T)], sem).wait()

    zero16 = jnp.zeros((L,), jnp.float32)
    lane = lax.iota(jnp.int32, L)
    _dnums = lax.GatherDimensionNumbers(
        offset_dims=(), collapsed_slice_dims=(0,), start_index_map=(0,))
    perms = [jnp.bitwise_xor(lane, jnp.full((L,), sh, jnp.int32))[:, None]
             for sh in (8, 4, 2, 1)]

    def hsum_all(v):
        for p in perms:
            v = v + lax.gather(v, p, _dnums, slice_sizes=(1,),
                               mode=lax.GatherScatterMode.PROMISE_IN_BOUNDS)
        return v

    def compute_chunk(rows, outb):
        @plsc.parallel_loop(0, T // 2)
        def tbody(u):
            tt = 2 * u

            def sum6(ti, off):
                r0 = rows[ti, pl.ds(off, L)]
                r1 = rows[T + ti, pl.ds(off, L)]
                r2 = rows[2 * T + ti, pl.ds(off, L)]
                r3 = rows[3 * T + ti, pl.ds(off, L)]
                r4 = rows[4 * T + ti, pl.ds(off, L)]
                r5 = rows[5 * T + ti, pl.ds(off, L)]
                v = ((r0 + r1) + (r2 + r3)) + (r4 + r5)
                outb[ti, pl.ds(off, L)] = v
                return v

            @plsc.parallel_loop(0, NJ // 2, carry=(zero16, zero16, zero16, zero16),
                                unroll=4)
            def pass1(j, carry):
                vs0, vq0, vs1, vq1 = carry
                off = j * (2 * L)
                a0 = sum6(tt, off)
                b0 = sum6(tt, off + L)
                a1 = sum6(tt + 1, off)
                b1 = sum6(tt + 1, off + L)
                return (vs0 + (a0 + b0), vq0 + (a0 * a0 + b0 * b0),
                        vs1 + (a1 + b1), vq1 + (a1 * a1 + b1 * b1))

            vs0, vq0, vs1, vq1 = pass1

            def stats(vs, vq):
                mv = hsum_all(vs) * (1.0 / D)
                av = hsum_all(vq) * (1.0 / D) - mv * mv + EPS
                ai = lax.bitcast_convert_type(av, jnp.int32)
                yi = jnp.full((L,), 0x5F3759DF, jnp.int32) - lax.shift_right_logical(
                    ai, jnp.full((L,), 1, jnp.int32))
                y = lax.bitcast_convert_type(yi, jnp.float32)
                ha = av * 0.5
                y = y * (1.5 - ha * y * y)
                y = y * (1.5 - ha * y * y)
                y = y * (1.5 - ha * y * y)
                return mv, y

            mv0, yv0 = stats(vs0, vq0)
            mv1, yv1 = stats(vs1, vq1)

            @plsc.parallel_loop(0, NJ // 2, unroll=4)
            def pass2(j):
                off = j * (2 * L)
                for oo in (off, off + L):
                    wv = w_v[pl.ds(oo, L)]
                    bv = b_v[pl.ds(oo, L)]
                    v0 = (outb[tt, pl.ds(oo, L)] - mv0) * yv0
                    outb[tt, pl.ds(oo, L)] = v0 * wv + bv
                    v1 = (outb[tt + 1, pl.ds(oo, L)] - mv1) * yv1
                    outb[tt + 1, pl.ds(oo, L)] = v1 * wv + bv

    bufs = ((rows0, out0, gsem0, osem0), (rows1, out1, gsem1, osem1))

    gather_start(0, rows0, gsem0)
    gather_start(1, rows1, gsem1)
    for bb in range(2):
        rows, outb, gsem, osem = bufs[bb]
        gather_wait(bb, rows, gsem)
        compute_chunk(rows, outb)
        out_start(bb, outb, osem)
        gather_start(2 + bb, rows, gsem)

    def ccbody(cc, _):
        for bb in range(2):
            rows, outb, gsem, osem = bufs[bb]
            c = 2 * cc + bb
            gather_wait(c, rows, gsem)
            out_wait(c - 2, outb, osem)
            compute_chunk(rows, outb)
            out_start(c, outb, osem)

            @pl.when(c + 2 < NCHUNK)
            def _():
                gather_start(c + 2, rows, gsem)
        return 0

    lax.fori_loop(1, NCHUNK // 2, ccbody, 0)

    out_wait(NCHUNK - 2, out0, osem0)
    out_wait(NCHUNK - 1, out1, osem1)


@functools.partial(jax.jit, static_argnames=())
def _sc_call(table, bboxT, w, b):
    mesh = plsc.VectorSubcoreMesh(core_axis_name="c", subcore_axis_name="s")
    return pl.kernel(
        _sc_body,
        out_type=jax.ShapeDtypeStruct((N, D), jnp.float32),
        mesh=mesh,
        scratch_types=[
            pltpu.VMEM((4, NTOK), jnp.int32),
            pltpu.VMEM((6, NTOK), jnp.int32),
            pltpu.VMEM((RPC, D), jnp.float32),
            pltpu.VMEM((RPC, D), jnp.float32),
            pltpu.VMEM((T, D), jnp.float32),
            pltpu.VMEM((T, D), jnp.float32),
            pltpu.VMEM((D,), jnp.float32),
            pltpu.VMEM((D,), jnp.float32),
            pltpu.SemaphoreType.DMA,
            pltpu.SemaphoreType.DMA,
            pltpu.SemaphoreType.DMA,
            pltpu.SemaphoreType.DMA,
        ],
    )(table, bboxT, w, b)


def kernel(bbox, x_table, y_table, h_table, w_table, ln_weight, ln_bias):
    table = jnp.concatenate([x_table, y_table, h_table, w_table], axis=0)
    bboxT = bbox.reshape(N, 4).T.astype(jnp.int32)
    out = _sc_call(table, bboxT, ln_weight, ln_bias)
    return out.reshape(B, S, D)


# speedup vs baseline: 1.5072x; 1.5072x over previous
import functools

import jax
import jax.numpy as jnp
from jax import lax
from jax.experimental import pallas as pl
from jax.experimental.pallas import tpu as pltpu
from jax.experimental.pallas import tpu_sc as plsc

B, S = 4, 2048
V, D = 1024, 768
EPS = 1e-12

NC, NS, L = 2, 16, 16
NW = NC * NS
N = B * S
NTOK = N // NW
T = 8
NCHUNK = NTOK // T
RPC = 6 * T
NJ = D // L


def _sc_body(table_hbm, bboxT_hbm, w_hbm, b_hbm, out_hbm,
             bbox_v, idx_v, rows0, rows1, out0, out1, w_v, b_v,
             gsem0, gsem1, osem0, osem1):
    wid = lax.axis_index("s") * NC + lax.axis_index("c")
    base = wid * NTOK

    for g in range(4):
        pltpu.sync_copy(bboxT_hbm.at[g, pl.ds(base, NTOK)], bbox_v.at[g])
    pltpu.sync_copy(w_hbm, w_v)
    pltpu.sync_copy(b_hbm, b_v)

    for i in range(NTOK // L):
        t0 = i * L
        b0 = bbox_v[0, pl.ds(t0, L)]
        b1 = bbox_v[1, pl.ds(t0, L)]
        b2 = bbox_v[2, pl.ds(t0, L)]
        b3 = bbox_v[3, pl.ds(t0, L)]
        vals = (b0, b1 + V, b2, b3 + V, (b3 - b1) + 2 * V, (b2 - b0) + 3 * V)
        for g in range(6):
            idx_v[g, pl.ds(t0, L)] = vals[g]

    def gather_start(c, rows, sem):
        for g in range(6):
            pltpu.make_async_copy(
                table_hbm.at[idx_v.at[g, pl.ds(c * T, T)]],
                rows.at[pl.ds(g * T, T)], sem).start()

    def gather_wait(c, rows, sem):
        for g in range(6):
            pltpu.make_async_copy(
                table_hbm.at[idx_v.at[g, pl.ds(c * T, T)]],
                rows.at[pl.ds(g * T, T)], sem).wait()

    def out_start(c, outb, sem):
        pltpu.make_async_copy(
            outb, out_hbm.at[pl.ds(base + c * T, T)], sem).start()

    def out_wait(c, outb, sem):
        pltpu.make_async_copy(
            outb, out_hbm.at[pl.ds(base + c * T, T)], sem).wait()

    zero16 = jnp.zeros((L,), jnp.float32)
    lane = lax.iota(jnp.int32, L)
    _dnums = lax.GatherDimensionNumbers(
        offset_dims=(), collapsed_slice_dims=(0,), start_index_map=(0,))
    perms = [jnp.bitwise_xor(lane, jnp.full((L,), sh, jnp.int32))[:, None]
             for sh in (8, 4, 2, 1)]

    def hsum_all(v):
        for p in perms:
            v = v + lax.gather(v, p, _dnums, slice_sizes=(1,),
                               mode=lax.GatherScatterMode.PROMISE_IN_BOUNDS)
        return v

    def stats(vs, vq):
        mv = hsum_all(vs) * (1.0 / D)
        av = hsum_all(vq) * (1.0 / D) - mv * mv + EPS
        ai = lax.bitcast_convert_type(av, jnp.int32)
        yi = jnp.full((L,), 0x5F3759DF, jnp.int32) - lax.shift_right_logical(
            ai, jnp.full((L,), 1, jnp.int32))
        y = lax.bitcast_convert_type(yi, jnp.float32)
        ha = av * 0.5
        y = y * (1.5 - ha * y * y)
        y = y * (1.5 - ha * y * y)
        y = y * (1.5 - ha * y * y)
        return mv, y

    def compute_chunk(rows, outb):
        def sum6(ti, off):
            r0 = rows[ti, pl.ds(off, L)]
            r1 = rows[T + ti, pl.ds(off, L)]
            r2 = rows[2 * T + ti, pl.ds(off, L)]
            r3 = rows[3 * T + ti, pl.ds(off, L)]
            r4 = rows[4 * T + ti, pl.ds(off, L)]
            r5 = rows[5 * T + ti, pl.ds(off, L)]
            v = ((r0 + r1) + (r2 + r3)) + (r4 + r5)
            outb[ti, pl.ds(off, L)] = v
            return v

        @plsc.parallel_loop(0, NJ, carry=(zero16,) * (2 * T), unroll=2)
        def pass1(j, carry):
            accs = list(carry)
            off = j * L
            for t in range(T):
                v = sum6(t, off)
                accs[t] = accs[t] + v
                accs[T + t] = accs[T + t] + v * v
            return tuple(accs)

        accs = pass1
        mys = [stats(accs[t], accs[T + t]) for t in range(T)]

        @plsc.parallel_loop(0, NJ, unroll=2)
        def pass2(j):
            oo = j * L
            wv = w_v[pl.ds(oo, L)]
            bv = b_v[pl.ds(oo, L)]
            for t in range(T):
                mv, yv = mys[t]
                v = (outb[t, pl.ds(oo, L)] - mv) * yv
                outb[t, pl.ds(oo, L)] = v * wv + bv

    bufs = ((rows0, out0, gsem0, osem0), (rows1, out1, gsem1, osem1))

    gather_start(0, rows0, gsem0)
    gather_start(1, rows1, gsem1)
    for bb in range(2):
        rows, outb, gsem, osem = bufs[bb]
        gather_wait(bb, rows, gsem)
        compute_chunk(rows, outb)
        out_start(bb, outb, osem)
        gather_start(2 + bb, rows, gsem)

    def ccbody(cc, _):
        for bb in range(2):
            rows, outb, gsem, osem = bufs[bb]
            c = 2 * cc + bb
            gather_wait(c, rows, gsem)
            out_wait(c - 2, outb, osem)
            compute_chunk(rows, outb)
            out_start(c, outb, osem)

            @pl.when(c + 2 < NCHUNK)
            def _():
                gather_start(c + 2, rows, gsem)
        return 0

    lax.fori_loop(1, NCHUNK // 2, ccbody, 0)

    out_wait(NCHUNK - 2, out0, osem0)
    out_wait(NCHUNK - 1, out1, osem1)


@functools.partial(jax.jit, static_argnames=())
def _sc_call(table, bboxT, w, b):
    mesh = plsc.VectorSubcoreMesh(core_axis_name="c", subcore_axis_name="s")
    return pl.kernel(
        _sc_body,
        out_type=jax.ShapeDtypeStruct((N, D), jnp.float32),
        mesh=mesh,
        scratch_types=[
            pltpu.VMEM((4, NTOK), jnp.int32),
            pltpu.VMEM((6, NTOK), jnp.int32),
            pltpu.VMEM((RPC, D), jnp.float32),
            pltpu.VMEM((RPC, D), jnp.float32),
            pltpu.VMEM((T, D), jnp.float32),
            pltpu.VMEM((T, D), jnp.float32),
            pltpu.VMEM((D,), jnp.float32),
            pltpu.VMEM((D,), jnp.float32),
            pltpu.SemaphoreType.DMA,
            pltpu.SemaphoreType.DMA,
            pltpu.SemaphoreType.DMA,
            pltpu.SemaphoreType.DMA,
        ],
    )(table, bboxT, w, b)


def kernel(bbox, x_table, y_table, h_table, w_table, ln_weight, ln_bias):
    table = jnp.concatenate([x_table, y_table, h_table, w_table], axis=0)
    bboxT = bbox.reshape(N, 4).T.astype(jnp.int32)
    out = _sc_call(table, bboxT, ln_weight, ln_bias)
    return out.reshape(B, S, D)


# speedup vs baseline: 1.5856x; 1.0520x over previous
import functools

import jax
import jax.numpy as jnp
from jax import lax
from jax.experimental import pallas as pl
from jax.experimental.pallas import tpu as pltpu
from jax.experimental.pallas import tpu_sc as plsc

B, S = 4, 2048
V, D = 1024, 768
EPS = 1e-12

NC, NS, L = 2, 16, 16
NW = NC * NS
N = B * S
NTOK = N // NW
T = 8
NCHUNK = NTOK // T
RPC = 6 * T
NJ = D // L


def _sc_body(xt_hbm, yt_hbm, ht_hbm, wt_hbm, bboxT_hbm, w_hbm, b_hbm, out_hbm,
             bbox_v, idx_v, rows0, rows1, out0, out1, w_v, b_v,
             gsem0, gsem1, osem0, osem1):
    tables = (xt_hbm, yt_hbm, xt_hbm, yt_hbm, ht_hbm, wt_hbm)
    wid = lax.axis_index("s") * NC + lax.axis_index("c")
    base = wid * NTOK

    for g in range(4):
        pltpu.sync_copy(bboxT_hbm.at[g, pl.ds(base, NTOK)], bbox_v.at[g])
    pltpu.sync_copy(w_hbm, w_v)
    pltpu.sync_copy(b_hbm, b_v)

    for i in range(NTOK // L):
        t0 = i * L
        b0 = bbox_v[0, pl.ds(t0, L)]
        b1 = bbox_v[1, pl.ds(t0, L)]
        b2 = bbox_v[2, pl.ds(t0, L)]
        b3 = bbox_v[3, pl.ds(t0, L)]
        vals = (b0, b1, b2, b3, b3 - b1, b2 - b0)
        for g in range(6):
            idx_v[g, pl.ds(t0, L)] = vals[g]

    def gather_start(c, rows, sem):
        for g in range(6):
            pltpu.make_async_copy(
                tables[g].at[idx_v.at[g, pl.ds(c * T, T)]],
                rows.at[pl.ds(g * T, T)], sem).start()

    def gather_wait(c, rows, sem):
        for g in range(6):
            pltpu.make_async_copy(
                tables[g].at[idx_v.at[g, pl.ds(c * T, T)]],
                rows.at[pl.ds(g * T, T)], sem).wait()

    def out_start(c, outb, sem):
        pltpu.make_async_copy(
            outb, out_hbm.at[pl.ds(base + c * T, T)], sem).start()

    def out_wait(c, outb, sem):
        pltpu.make_async_copy(
            outb, out_hbm.at[pl.ds(base + c * T, T)], sem).wait()

    zero16 = jnp.zeros((L,), jnp.float32)
    lane = lax.iota(jnp.int32, L)
    _dnums = lax.GatherDimensionNumbers(
        offset_dims=(), collapsed_slice_dims=(0,), start_index_map=(0,))
    perms = [jnp.bitwise_xor(lane, jnp.full((L,), sh, jnp.int32))[:, None]
             for sh in (8, 4, 2, 1)]

    def hsum_all(v):
        for p in perms:
            v = v + lax.gather(v, p, _dnums, slice_sizes=(1,),
                               mode=lax.GatherScatterMode.PROMISE_IN_BOUNDS)
        return v

    def stats(vs, vq):
        mv = hsum_all(vs) * (1.0 / D)
        av = hsum_all(vq) * (1.0 / D) - mv * mv + EPS
        ai = lax.bitcast_convert_type(av, jnp.int32)
        yi = jnp.full((L,), 0x5F3759DF, jnp.int32) - lax.shift_right_logical(
            ai, jnp.full((L,), 1, jnp.int32))
        y = lax.bitcast_convert_type(yi, jnp.float32)
        ha = av * 0.5
        y = y * (1.5 - ha * y * y)
        y = y * (1.5 - ha * y * y)
        y = y * (1.5 - ha * y * y)
        return mv, y

    def compute_chunk(rows, outb):
        def sum6(ti, off):
            r0 = rows[ti, pl.ds(off, L)]
            r1 = rows[T + ti, pl.ds(off, L)]
            r2 = rows[2 * T + ti, pl.ds(off, L)]
            r3 = rows[3 * T + ti, pl.ds(off, L)]
            r4 = rows[4 * T + ti, pl.ds(off, L)]
            r5 = rows[5 * T + ti, pl.ds(off, L)]
            v = ((r0 + r1) + (r2 + r3)) + (r4 + r5)
            outb[ti, pl.ds(off, L)] = v
            return v

        @plsc.parallel_loop(0, NJ, carry=(zero16,) * (2 * T), unroll=2)
        def pass1(j, carry):
            accs = list(carry)
            off = j * L
            for t in range(T):
                v = sum6(t, off)
                accs[t] = accs[t] + v
                accs[T + t] = accs[T + t] + v * v
            return tuple(accs)

        accs = pass1
        mys = [stats(accs[t], accs[T + t]) for t in range(T)]

        @plsc.parallel_loop(0, NJ, unroll=2)
        def pass2(j):
            oo = j * L
            wv = w_v[pl.ds(oo, L)]
            bv = b_v[pl.ds(oo, L)]
            for t in range(T):
                mv, yv = mys[t]
                v = (outb[t, pl.ds(oo, L)] - mv) * yv
                outb[t, pl.ds(oo, L)] = v * wv + bv

    bufs = ((rows0, out0, gsem0, osem0), (rows1, out1, gsem1, osem1))

    gather_start(0, rows0, gsem0)
    gather_start(1, rows1, gsem1)
    for bb in range(2):
        rows, outb, gsem, osem = bufs[bb]
        gather_wait(bb, rows, gsem)
        compute_chunk(rows, outb)
        out_start(bb, outb, osem)
        gather_start(2 + bb, rows, gsem)

    def ccbody(cc, _):
        for bb in range(2):
            rows, outb, gsem, osem = bufs[bb]
            c = 2 * cc + bb
            gather_wait(c, rows, gsem)
            out_wait(c - 2, outb, osem)
            compute_chunk(rows, outb)
            out_start(c, outb, osem)

            @pl.when(c + 2 < NCHUNK)
            def _():
                gather_start(c + 2, rows, gsem)
        return 0

    lax.fori_loop(1, NCHUNK // 2, ccbody, 0)

    out_wait(NCHUNK - 2, out0, osem0)
    out_wait(NCHUNK - 1, out1, osem1)


@functools.partial(jax.jit, static_argnames=())
def _sc_call(xt, yt, ht, wt, bboxT, w, b):
    mesh = plsc.VectorSubcoreMesh(core_axis_name="c", subcore_axis_name="s")
    return pl.kernel(
        _sc_body,
        out_type=jax.ShapeDtypeStruct((N, D), jnp.float32),
        mesh=mesh,
        scratch_types=[
            pltpu.VMEM((4, NTOK), jnp.int32),
            pltpu.VMEM((6, NTOK), jnp.int32),
            pltpu.VMEM((RPC, D), jnp.float32),
            pltpu.VMEM((RPC, D), jnp.float32),
            pltpu.VMEM((T, D), jnp.float32),
            pltpu.VMEM((T, D), jnp.float32),
            pltpu.VMEM((D,), jnp.float32),
            pltpu.VMEM((D,), jnp.float32),
            pltpu.SemaphoreType.DMA,
            pltpu.SemaphoreType.DMA,
            pltpu.SemaphoreType.DMA,
            pltpu.SemaphoreType.DMA,
        ],
    )(xt, yt, ht, wt, bboxT, w, b)


def kernel(bbox, x_table, y_table, h_table, w_table, ln_weight, ln_bias):
    bboxT = bbox.reshape(N, 4).T.astype(jnp.int32)
    out = _sc_call(x_table, y_table, h_table, w_table, bboxT,
                   ln_weight, ln_bias)
    return out.reshape(B, S, D)
